# baseline (speedup 1.0000x reference)
import jax
import jax.numpy as jnp
from jax import lax
from jax.experimental import pallas as pl
from jax.experimental.pallas import tpu as pltpu
from jax.experimental.pallas import tpu_sc as plsc

_S = 256
_K = _S * (_S + 1) // 2
_B = 4096
_NC = 2
_NS = 16
_NW = _NC * _NS
_RPW = _B // _NW
_XPAD = 528
_OPAD = _K + 272


def _off_py(i: int) -> int:
    return _S * i - i * (i - 1) // 2


def _compute_row(x_v, o_v):
    iota = lax.iota(jnp.int32, 16)

    for g in range(8):
        nch = 16 - g

        def seg(ii, carry, g=g, nch=nch):
            i = 16 * g + ii
            ivec = jnp.broadcast_to(i, (16,))
            left = plsc.load_gather(x_v, [ivec])
            base = (i << 8) - lax.shift_right_logical(i * (i - 1), 1)
            bvec = jnp.broadcast_to(base, (16,))
            for c in range(nch):
                cvec = iota + 16 * c
                right = plsc.load_gather(x_v, [ivec + cvec])
                prod = left * right
                if c == nch - 1:
                    plsc.store_scatter(
                        o_v, [bvec + cvec], prod, mask=iota < (16 - ii)
                    )
                else:
                    plsc.store_scatter(o_v, [bvec + cvec], prod)
            return carry

        lax.fori_loop(0, 16, seg, 0, unroll=2)

    for i0, nch, unroll in ((128, 8, 1), (192, 4, 2)):

        @plsc.parallel_loop(0, 64, 1, unroll=unroll)
        def seg(ii, i0=i0, nch=nch):
            i = i0 + ii
            ivec = jnp.broadcast_to(i, (16,))
            left = plsc.load_gather(x_v, [ivec])
            base = (i << 8) - lax.shift_right_logical(i * (i - 1), 1)
            bvec = jnp.broadcast_to(base, (16,))
            rem = _S - i
            for c in range(nch):
                cvec = iota + 16 * c
                right = plsc.load_gather(x_v, [ivec + cvec])
                plsc.store_scatter(
                    o_v, [bvec + cvec], left * right, mask=iota < (rem - 16 * c)
                )


def _sc_body(act_hbm, out_hbm, x0_v, x1_v, o0_v, o1_v,
             isem0, isem1, osem0, osem1):
    xs = (x0_v, x1_v)
    os_ = (o0_v, o1_v)
    isems = (isem0, isem1)
    osems = (osem0, osem1)
    wid = lax.axis_index("s") * _NC + lax.axis_index("c")
    row0 = wid * _RPW

    pltpu.async_copy(
        act_hbm.at[row0, pl.ds(0, _S)], xs[0].at[pl.ds(0, _S)], isems[0]
    )

    def body(r2, carry):
        for s in (0, 1):
            r = 2 * r2 + s
            row = row0 + r

            @pl.when(r < _RPW - 1)
            def _():
                pltpu.async_copy(
                    act_hbm.at[row + 1, pl.ds(0, _S)],
                    xs[1 - s].at[pl.ds(0, _S)],
                    isems[1 - s],
                )

            pltpu.make_async_copy(
                act_hbm.at[row, pl.ds(0, _S)], xs[s].at[pl.ds(0, _S)], isems[s]
            ).wait()

            @pl.when(r2 >= 1)
            def _():
                pltpu.make_async_copy(
                    os_[s].at[pl.ds(0, _K)], out_hbm.at[row - 2], osems[s]
                ).wait()

            _compute_row(xs[s], os_[s])

            pltpu.async_copy(os_[s].at[pl.ds(0, _K)], out_hbm.at[row], osems[s])
        return carry

    lax.fori_loop(0, _RPW // 2, body, 0)

    last = row0 + _RPW - 1
    pltpu.make_async_copy(
        os_[0].at[pl.ds(0, _K)], out_hbm.at[last - 1], osems[0]
    ).wait()
    pltpu.make_async_copy(
        os_[1].at[pl.ds(0, _K)], out_hbm.at[last], osems[1]
    ).wait()


@jax.jit
def _sc_pairing(activations):
    mesh = plsc.VectorSubcoreMesh(core_axis_name="c", subcore_axis_name="s")
    return pl.kernel(
        _sc_body,
        mesh=mesh,
        compiler_params=pltpu.CompilerParams(needs_layout_passes=False),
        out_type=jax.ShapeDtypeStruct((_B, _K), jnp.float32),
        scratch_types=[
            pltpu.VMEM((_XPAD,), jnp.float32),
            pltpu.VMEM((_XPAD,), jnp.float32),
            pltpu.VMEM((_OPAD,), jnp.float32),
            pltpu.VMEM((_OPAD,), jnp.float32),
            pltpu.SemaphoreType.DMA,
            pltpu.SemaphoreType.DMA,
            pltpu.SemaphoreType.DMA,
            pltpu.SemaphoreType.DMA,
        ],
    )(activations)


def kernel(activations):
    return _sc_pairing(activations)


# speedup vs baseline: 3.0867x; 3.0867x over previous
import jax
import jax.numpy as jnp
from jax import lax
from jax.experimental import pallas as pl
from jax.experimental.pallas import tpu as pltpu
from jax.experimental.pallas import tpu_sc as plsc

_S = 256
_K = _S * (_S + 1) // 2
_B = 4096
_NC = 2
_NS = 16
_NW = _NC * _NS
_RPW = _B // _NW
_XPAD = 528
_OPAD = _K + 272


def _off_py(i: int) -> int:
    return _S * i - i * (i - 1) // 2


def _compute_row(x_v, o_v):
    iota = lax.iota(jnp.int32, 16)

    for i0, nch in ((0, 16), (32, 14), (64, 12), (96, 10)):
        full = (_S - (i0 + 31)) // 16

        @plsc.parallel_loop(0, 32, 1)
        def seg(ii, i0=i0, nch=nch, full=full):
            i = i0 + ii
            ivec = jnp.broadcast_to(i, (16,))
            left = plsc.load_gather(x_v, [ivec])
            base = (i << 8) - lax.shift_right_logical(i * (i - 1), 1)
            bvec = jnp.broadcast_to(base, (16,))
            rem = _S - i
            for c in range(nch):
                cvec = iota + 16 * c
                right = plsc.load_gather(x_v, [ivec + cvec])
                prod = left * right
                if c < full:
                    plsc.store_scatter(o_v, [bvec + cvec], prod)
                else:
                    plsc.store_scatter(
                        o_v, [bvec + cvec], prod, mask=iota < (rem - 16 * c)
                    )

    for i0, nch, unroll in ((128, 8, 1), (192, 4, 2)):

        @plsc.parallel_loop(0, 64, 1, unroll=unroll)
        def seg(ii, i0=i0, nch=nch):
            i = i0 + ii
            ivec = jnp.broadcast_to(i, (16,))
            left = plsc.load_gather(x_v, [ivec])
            base = (i << 8) - lax.shift_right_logical(i * (i - 1), 1)
            bvec = jnp.broadcast_to(base, (16,))
            rem = _S - i
            for c in range(nch):
                cvec = iota + 16 * c
                right = plsc.load_gather(x_v, [ivec + cvec])
                plsc.store_scatter(
                    o_v, [bvec + cvec], left * right, mask=iota < (rem - 16 * c)
                )


def _sc_body(act_hbm, out_hbm, x0_v, x1_v, o0_v, o1_v,
             isem0, isem1, osem0, osem1):
    xs = (x0_v, x1_v)
    os_ = (o0_v, o1_v)
    isems = (isem0, isem1)
    osems = (osem0, osem1)
    wid = lax.axis_index("s") * _NC + lax.axis_index("c")
    row0 = wid * _RPW

    pltpu.async_copy(
        act_hbm.at[row0, pl.ds(0, _S)], xs[0].at[pl.ds(0, _S)], isems[0]
    )

    def body(r2, carry):
        for s in (0, 1):
            r = 2 * r2 + s
            row = row0 + r

            @pl.when(r < _RPW - 1)
            def _():
                pltpu.async_copy(
                    act_hbm.at[row + 1, pl.ds(0, _S)],
                    xs[1 - s].at[pl.ds(0, _S)],
                    isems[1 - s],
                )

            pltpu.make_async_copy(
                act_hbm.at[row, pl.ds(0, _S)], xs[s].at[pl.ds(0, _S)], isems[s]
            ).wait()

            @pl.when(r2 >= 1)
            def _():
                pltpu.make_async_copy(
                    os_[s].at[pl.ds(0, _K)], out_hbm.at[row - 2], osems[s]
                ).wait()

            _compute_row(xs[s], os_[s])

            pltpu.async_copy(os_[s].at[pl.ds(0, _K)], out_hbm.at[row], osems[s])
        return carry

    lax.fori_loop(0, _RPW // 2, body, 0)

    last = row0 + _RPW - 1
    pltpu.make_async_copy(
        os_[0].at[pl.ds(0, _K)], out_hbm.at[last - 1], osems[0]
    ).wait()
    pltpu.make_async_copy(
        os_[1].at[pl.ds(0, _K)], out_hbm.at[last], osems[1]
    ).wait()


@jax.jit
def _sc_pairing(activations):
    mesh = plsc.VectorSubcoreMesh(core_axis_name="c", subcore_axis_name="s")
    return pl.kernel(
        _sc_body,
        mesh=mesh,
        compiler_params=pltpu.CompilerParams(needs_layout_passes=False),
        out_type=jax.ShapeDtypeStruct((_B, _K), jnp.float32),
        scratch_types=[
            pltpu.VMEM((_XPAD,), jnp.float32),
            pltpu.VMEM((_XPAD,), jnp.float32),
            pltpu.VMEM((_OPAD,), jnp.float32),
            pltpu.VMEM((_OPAD,), jnp.float32),
            pltpu.SemaphoreType.DMA,
            pltpu.SemaphoreType.DMA,
            pltpu.SemaphoreType.DMA,
            pltpu.SemaphoreType.DMA,
        ],
    )(activations)


def kernel(activations):
    return _sc_pairing(activations)


# speedup vs baseline: 3.6870x; 1.1945x over previous
import jax
import jax.numpy as jnp
from jax import lax
from jax.experimental import pallas as pl
from jax.experimental.pallas import tpu as pltpu
from jax.experimental.pallas import tpu_sc as plsc

_S = 256
_K = _S * (_S + 1) // 2
_B = 4096
_NC = 2
_NS = 16
_NW = _NC * _NS
_RPW = _B // _NW
_XPAD = 528
_OPAD = _K + 272


def _off_py(i: int) -> int:
    return _S * i - i * (i - 1) // 2


def _compute_row(x_v, o_v):
    iota = lax.iota(jnp.int32, 16)

    for i0, nch in ((0, 16), (64, 12)):
        full = (_S - (i0 + 63)) // 16

        @plsc.parallel_loop(0, 64, 1)
        def seg(ii, i0=i0, nch=nch, full=full):
            i = i0 + ii
            ivec = jnp.broadcast_to(i, (16,))
            left = plsc.load_gather(x_v, [ivec])
            base = (i << 8) - lax.shift_right_logical(i * (i - 1), 1)
            bvec = jnp.broadcast_to(base, (16,))
            rem = _S - i
            for c in range(nch):
                cvec = iota + 16 * c
                right = plsc.load_gather(x_v, [ivec + cvec])
                prod = left * right
                if c < full:
                    plsc.store_scatter(o_v, [bvec + cvec], prod)
                else:
                    plsc.store_scatter(
                        o_v, [bvec + cvec], prod, mask=iota < (rem - 16 * c)
                    )

    for i0, nch, unroll in ((128, 8, 1), (192, 4, 2)):

        @plsc.parallel_loop(0, 64, 1, unroll=unroll)
        def seg(ii, i0=i0, nch=nch):
            i = i0 + ii
            ivec = jnp.broadcast_to(i, (16,))
            left = plsc.load_gather(x_v, [ivec])
            base = (i << 8) - lax.shift_right_logical(i * (i - 1), 1)
            bvec = jnp.broadcast_to(base, (16,))
            rem = _S - i
            for c in range(nch):
                cvec = iota + 16 * c
                right = plsc.load_gather(x_v, [ivec + cvec])
                plsc.store_scatter(
                    o_v, [bvec + cvec], left * right, mask=iota < (rem - 16 * c)
                )


def _sc_body(act_hbm, out_hbm, x0_v, x1_v, o0_v, o1_v,
             isem0, isem1, osem0, osem1):
    xs = (x0_v, x1_v)
    os_ = (o0_v, o1_v)
    isems = (isem0, isem1)
    osems = (osem0, osem1)
    wid = lax.axis_index("s") * _NC + lax.axis_index("c")
    row0 = wid * _RPW

    pltpu.async_copy(
        act_hbm.at[row0, pl.ds(0, _S)], xs[0].at[pl.ds(0, _S)], isems[0]
    )

    def body(r2, carry):
        for s in (0, 1):
            r = 2 * r2 + s
            row = row0 + r

            @pl.when(r < _RPW - 1)
            def _():
                pltpu.async_copy(
                    act_hbm.at[row + 1, pl.ds(0, _S)],
                    xs[1 - s].at[pl.ds(0, _S)],
                    isems[1 - s],
                )

            pltpu.make_async_copy(
                act_hbm.at[row, pl.ds(0, _S)], xs[s].at[pl.ds(0, _S)], isems[s]
            ).wait()

            @pl.when(r2 >= 1)
            def _():
                pltpu.make_async_copy(
                    os_[s].at[pl.ds(0, _K)], out_hbm.at[row - 2], osems[s]
                ).wait()

            _compute_row(xs[s], os_[s])

            pltpu.async_copy(os_[s].at[pl.ds(0, _K)], out_hbm.at[row], osems[s])
        return carry

    lax.fori_loop(0, _RPW // 2, body, 0)

    last = row0 + _RPW - 1
    pltpu.make_async_copy(
        os_[0].at[pl.ds(0, _K)], out_hbm.at[last - 1], osems[0]
    ).wait()
    pltpu.make_async_copy(
        os_[1].at[pl.ds(0, _K)], out_hbm.at[last], osems[1]
    ).wait()


@jax.jit
def _sc_pairing(activations):
    mesh = plsc.VectorSubcoreMesh(core_axis_name="c", subcore_axis_name="s")
    return pl.kernel(
        _sc_body,
        mesh=mesh,
        compiler_params=pltpu.CompilerParams(needs_layout_passes=False),
        out_type=jax.ShapeDtypeStruct((_B, _K), jnp.float32),
        scratch_types=[
            pltpu.VMEM((_XPAD,), jnp.float32),
            pltpu.VMEM((_XPAD,), jnp.float32),
            pltpu.VMEM((_OPAD,), jnp.float32),
            pltpu.VMEM((_OPAD,), jnp.float32),
            pltpu.SemaphoreType.DMA,
            pltpu.SemaphoreType.DMA,
            pltpu.SemaphoreType.DMA,
            pltpu.SemaphoreType.DMA,
        ],
    )(activations)


def kernel(activations):
    return _sc_pairing(activations)


# speedup vs baseline: 3.6979x; 1.0029x over previous
import jax
import jax.numpy as jnp
from jax import lax
from jax.experimental import pallas as pl
from jax.experimental.pallas import tpu as pltpu
from jax.experimental.pallas import tpu_sc as plsc

_S = 256
_K = _S * (_S + 1) // 2
_B = 4096
_NC = 2
_NS = 16
_NW = _NC * _NS
_RPW = _B // _NW
_XPAD = 528
_OPAD = _K + 272


def _off_py(i: int) -> int:
    return _S * i - i * (i - 1) // 2


def _compute_row(x_v, o_v):
    iota = lax.iota(jnp.int32, 16)

    for i0, nch in ((0, 16), (64, 12)):
        full = (_S - (i0 + 63)) // 16

        @plsc.parallel_loop(0, 64, 1)
        def seg(ii, i0=i0, nch=nch, full=full):
            i = i0 + ii
            ivec = jnp.broadcast_to(i, (16,))
            base = (i << 8) - lax.shift_right_logical(i * (i - 1), 1)
            bvec = jnp.broadcast_to(base, (16,))
            rem = _S - i
            left = None
            for c in range(nch):
                cvec = iota + 16 * c
                right = plsc.load_gather(x_v, [ivec + cvec])
                if left is None:
                    left = jnp.broadcast_to(right[0], (16,))
                prod = left * right
                if c < full:
                    plsc.store_scatter(o_v, [bvec + cvec], prod)
                else:
                    plsc.store_scatter(
                        o_v, [bvec + cvec], prod, mask=iota < (rem - 16 * c)
                    )

    for i0, nch, unroll in ((128, 8, 1), (192, 4, 2)):

        @plsc.parallel_loop(0, 64, 1, unroll=unroll)
        def seg(ii, i0=i0, nch=nch):
            i = i0 + ii
            ivec = jnp.broadcast_to(i, (16,))
            base = (i << 8) - lax.shift_right_logical(i * (i - 1), 1)
            bvec = jnp.broadcast_to(base, (16,))
            rem = _S - i
            left = None
            for c in range(nch):
                cvec = iota + 16 * c
                right = plsc.load_gather(x_v, [ivec + cvec])
                if left is None:
                    left = jnp.broadcast_to(right[0], (16,))
                plsc.store_scatter(
                    o_v, [bvec + cvec], left * right, mask=iota < (rem - 16 * c)
                )


def _sc_body(act_hbm, out_hbm, x0_v, x1_v, o0_v, o1_v,
             isem0, isem1, osem0, osem1):
    xs = (x0_v, x1_v)
    os_ = (o0_v, o1_v)
    isems = (isem0, isem1)
    osems = (osem0, osem1)
    wid = lax.axis_index("s") * _NC + lax.axis_index("c")
    row0 = wid * _RPW

    pltpu.async_copy(
        act_hbm.at[row0, pl.ds(0, _S)], xs[0].at[pl.ds(0, _S)], isems[0]
    )

    def body(r2, carry):
        for s in (0, 1):
            r = 2 * r2 + s
            row = row0 + r

            @pl.when(r < _RPW - 1)
            def _():
                pltpu.async_copy(
                    act_hbm.at[row + 1, pl.ds(0, _S)],
                    xs[1 - s].at[pl.ds(0, _S)],
                    isems[1 - s],
                )

            pltpu.make_async_copy(
                act_hbm.at[row, pl.ds(0, _S)], xs[s].at[pl.ds(0, _S)], isems[s]
            ).wait()

            @pl.when(r2 >= 1)
            def _():
                pltpu.make_async_copy(
                    os_[s].at[pl.ds(0, _K)], out_hbm.at[row - 2], osems[s]
                ).wait()

            _compute_row(xs[s], os_[s])

            pltpu.async_copy(os_[s].at[pl.ds(0, _K)], out_hbm.at[row], osems[s])
        return carry

    lax.fori_loop(0, _RPW // 2, body, 0)

    last = row0 + _RPW - 1
    pltpu.make_async_copy(
        os_[0].at[pl.ds(0, _K)], out_hbm.at[last - 1], osems[0]
    ).wait()
    pltpu.make_async_copy(
        os_[1].at[pl.ds(0, _K)], out_hbm.at[last], osems[1]
    ).wait()


@jax.jit
def _sc_pairing(activations):
    mesh = plsc.VectorSubcoreMesh(core_axis_name="c", subcore_axis_name="s")
    return pl.kernel(
        _sc_body,
        mesh=mesh,
        compiler_params=pltpu.CompilerParams(needs_layout_passes=False),
        out_type=jax.ShapeDtypeStruct((_B, _K), jnp.float32),
        scratch_types=[
            pltpu.VMEM((_XPAD,), jnp.float32),
            pltpu.VMEM((_XPAD,), jnp.float32),
            pltpu.VMEM((_OPAD,), jnp.float32),
            pltpu.VMEM((_OPAD,), jnp.float32),
            pltpu.SemaphoreType.DMA,
            pltpu.SemaphoreType.DMA,
            pltpu.SemaphoreType.DMA,
            pltpu.SemaphoreType.DMA,
        ],
    )(activations)


def kernel(activations):
    return _sc_pairing(activations)


# speedup vs baseline: 3.7606x; 1.0170x over previous
import jax
import jax.numpy as jnp
from jax import lax
from jax.experimental import pallas as pl
from jax.experimental.pallas import tpu as pltpu
from jax.experimental.pallas import tpu_sc as plsc

_S = 256
_K = _S * (_S + 1) // 2
_B = 4096
_NC = 2
_NS = 16
_NW = _NC * _NS
_RPW = _B // _NW
_XPAD = 528
_OPAD = _K + 272


def _off_py(i: int) -> int:
    return _S * i - i * (i - 1) // 2


def _compute_row(x_v, o_v):
    iota = lax.iota(jnp.int32, 16)

    for i0, nch in ((0, 16), (64, 12)):
        full = (_S - (i0 + 63)) // 16

        @plsc.parallel_loop(0, 64, 1)
        def seg(ii, i0=i0, nch=nch, full=full):
            i = i0 + ii
            ivec = jnp.broadcast_to(i, (16,))
            base = (i << 8) - lax.shift_right_logical(i * (i - 1), 1)
            bvec = jnp.broadcast_to(base, (16,))
            rem = _S - i
            left = None
            for c in range(nch):
                cvec = iota + 16 * c
                right = plsc.load_gather(x_v, [ivec + cvec])
                if left is None:
                    left = jnp.broadcast_to(right[0], (16,))
                prod = left * right
                if c < full:
                    plsc.store_scatter(o_v, [bvec + cvec], prod)
                else:
                    plsc.store_scatter(
                        o_v, [bvec + cvec], prod, mask=iota < (rem - 16 * c)
                    )

    for i0, nch, unroll in ((128, 8, 2), (192, 4, 2)):

        @plsc.parallel_loop(0, 64, 1, unroll=unroll)
        def seg(ii, i0=i0, nch=nch):
            i = i0 + ii
            ivec = jnp.broadcast_to(i, (16,))
            base = (i << 8) - lax.shift_right_logical(i * (i - 1), 1)
            bvec = jnp.broadcast_to(base, (16,))
            rem = _S - i
            left = None
            for c in range(nch):
                cvec = iota + 16 * c
                right = plsc.load_gather(x_v, [ivec + cvec])
                if left is None:
                    left = jnp.broadcast_to(right[0], (16,))
                plsc.store_scatter(
                    o_v, [bvec + cvec], left * right, mask=iota < (rem - 16 * c)
                )


def _sc_body(act_hbm, out_hbm, x0_v, x1_v, o0_v, o1_v,
             isem0, isem1, osem0, osem1):
    xs = (x0_v, x1_v)
    os_ = (o0_v, o1_v)
    isems = (isem0, isem1)
    osems = (osem0, osem1)
    wid = lax.axis_index("s") * _NC + lax.axis_index("c")
    row0 = wid * _RPW

    pltpu.async_copy(
        act_hbm.at[row0, pl.ds(0, _S)], xs[0].at[pl.ds(0, _S)], isems[0]
    )

    def body(r2, carry):
        for s in (0, 1):
            r = 2 * r2 + s
            row = row0 + r

            @pl.when(r < _RPW - 1)
            def _():
                pltpu.async_copy(
                    act_hbm.at[row + 1, pl.ds(0, _S)],
                    xs[1 - s].at[pl.ds(0, _S)],
                    isems[1 - s],
                )

            pltpu.make_async_copy(
                act_hbm.at[row, pl.ds(0, _S)], xs[s].at[pl.ds(0, _S)], isems[s]
            ).wait()

            @pl.when(r2 >= 1)
            def _():
                pltpu.make_async_copy(
                    os_[s].at[pl.ds(0, _K)], out_hbm.at[row - 2], osems[s]
                ).wait()

            _compute_row(xs[s], os_[s])

            pltpu.async_copy(os_[s].at[pl.ds(0, _K)], out_hbm.at[row], osems[s])
        return carry

    lax.fori_loop(0, _RPW // 2, body, 0)

    last = row0 + _RPW - 1
    pltpu.make_async_copy(
        os_[0].at[pl.ds(0, _K)], out_hbm.at[last - 1], osems[0]
    ).wait()
    pltpu.make_async_copy(
        os_[1].at[pl.ds(0, _K)], out_hbm.at[last], osems[1]
    ).wait()


@jax.jit
def _sc_pairing(activations):
    mesh = plsc.VectorSubcoreMesh(core_axis_name="c", subcore_axis_name="s")
    return pl.kernel(
        _sc_body,
        mesh=mesh,
        compiler_params=pltpu.CompilerParams(needs_layout_passes=False),
        out_type=jax.ShapeDtypeStruct((_B, _K), jnp.float32),
        scratch_types=[
            pltpu.VMEM((_XPAD,), jnp.float32),
            pltpu.VMEM((_XPAD,), jnp.float32),
            pltpu.VMEM((_OPAD,), jnp.float32),
            pltpu.VMEM((_OPAD,), jnp.float32),
            pltpu.SemaphoreType.DMA,
            pltpu.SemaphoreType.DMA,
            pltpu.SemaphoreType.DMA,
            pltpu.SemaphoreType.DMA,
        ],
    )(activations)


def kernel(activations):
    return _sc_pairing(activations)


# speedup vs baseline: 3.8482x; 1.0233x over previous
import jax
import jax.numpy as jnp
from jax import lax
from jax.experimental import pallas as pl
from jax.experimental.pallas import tpu as pltpu
from jax.experimental.pallas import tpu_sc as plsc

_S = 256
_K = _S * (_S + 1) // 2
_B = 4096
_NC = 2
_NS = 16
_NW = _NC * _NS
_RPW = _B // _NW
_XPAD = 528
_OPAD = _K + 272


def _off_py(i: int) -> int:
    return _S * i - i * (i - 1) // 2


def _compute_row(x_v, o_v):
    iota = lax.iota(jnp.int32, 16)

    for i0, nch in ((0, 16), (64, 12)):
        full = (_S - (i0 + 63)) // 16

        @plsc.parallel_loop(0, 64, 1)
        def seg(ii, i0=i0, nch=nch, full=full):
            i = i0 + ii
            ivec = jnp.broadcast_to(i, (16,))
            base = (i << 8) - lax.shift_right_logical(i * (i - 1), 1)
            bvec = jnp.broadcast_to(base, (16,))
            rem = _S - i
            left = None
            for c in range(nch):
                cvec = iota + 16 * c
                right = plsc.load_gather(x_v, [ivec + cvec])
                if left is None:
                    left = jnp.broadcast_to(right[0], (16,))
                prod = left * right
                if c < full:
                    plsc.store_scatter(o_v, [bvec + cvec], prod)
                else:
                    plsc.store_scatter(
                        o_v, [bvec + cvec], prod, mask=iota < (rem - 16 * c)
                    )

    for i0, nch, unroll in ((128, 8, 2), (192, 4, 4)):

        @plsc.parallel_loop(0, 64, 1, unroll=unroll)
        def seg(ii, i0=i0, nch=nch):
            i = i0 + ii
            ivec = jnp.broadcast_to(i, (16,))
            base = (i << 8) - lax.shift_right_logical(i * (i - 1), 1)
            bvec = jnp.broadcast_to(base, (16,))
            rem = _S - i
            left = None
            for c in range(nch):
                cvec = iota + 16 * c
                right = plsc.load_gather(x_v, [ivec + cvec])
                if left is None:
                    left = jnp.broadcast_to(right[0], (16,))
                plsc.store_scatter(
                    o_v, [bvec + cvec], left * right, mask=iota < (rem - 16 * c)
                )


def _sc_body(act_hbm, out_hbm, x0_v, x1_v, o0_v, o1_v,
             isem0, isem1, osem0, osem1):
    xs = (x0_v, x1_v)
    os_ = (o0_v, o1_v)
    isems = (isem0, isem1)
    osems = (osem0, osem1)
    wid = lax.axis_index("s") * _NC + lax.axis_index("c")
    row0 = wid * _RPW

    pltpu.async_copy(
        act_hbm.at[row0, pl.ds(0, _S)], xs[0].at[pl.ds(0, _S)], isems[0]
    )

    def body(r2, carry):
        for s in (0, 1):
            r = 2 * r2 + s
            row = row0 + r

            @pl.when(r < _RPW - 1)
            def _():
                pltpu.async_copy(
                    act_hbm.at[row + 1, pl.ds(0, _S)],
                    xs[1 - s].at[pl.ds(0, _S)],
                    isems[1 - s],
                )

            pltpu.make_async_copy(
                act_hbm.at[row, pl.ds(0, _S)], xs[s].at[pl.ds(0, _S)], isems[s]
            ).wait()

            @pl.when(r2 >= 1)
            def _():
                pltpu.make_async_copy(
                    os_[s].at[pl.ds(0, _K)], out_hbm.at[row - 2], osems[s]
                ).wait()

            _compute_row(xs[s], os_[s])

            pltpu.async_copy(os_[s].at[pl.ds(0, _K)], out_hbm.at[row], osems[s])
        return carry

    lax.fori_loop(0, _RPW // 2, body, 0)

    last = row0 + _RPW - 1
    pltpu.make_async_copy(
        os_[0].at[pl.ds(0, _K)], out_hbm.at[last - 1], osems[0]
    ).wait()
    pltpu.make_async_copy(
        os_[1].at[pl.ds(0, _K)], out_hbm.at[last], osems[1]
    ).wait()


@jax.jit
def _sc_pairing(activations):
    mesh = plsc.VectorSubcoreMesh(core_axis_name="c", subcore_axis_name="s")
    return pl.kernel(
        _sc_body,
        mesh=mesh,
        compiler_params=pltpu.CompilerParams(needs_layout_passes=False),
        out_type=jax.ShapeDtypeStruct((_B, _K), jnp.float32),
        scratch_types=[
            pltpu.VMEM((_XPAD,), jnp.float32),
            pltpu.VMEM((_XPAD,), jnp.float32),
            pltpu.VMEM((_OPAD,), jnp.float32),
            pltpu.VMEM((_OPAD,), jnp.float32),
            pltpu.SemaphoreType.DMA,
            pltpu.SemaphoreType.DMA,
            pltpu.SemaphoreType.DMA,
            pltpu.SemaphoreType.DMA,
        ],
    )(activations)


def kernel(activations):
    return _sc_pairing(activations)
